# baseline (speedup 1.0000x reference)
import dataclasses
import functools

import jax
import jax.numpy as jnp
from jax import lax
from jax.experimental import pallas as pl
from jax.experimental.pallas import tpu as pltpu
from jax.experimental.pallas import tpu_sc as plsc

_PH, _PW = 7, 7
_NBINS = _PH * _PW
_LANES = 16
_NWORKERS = 32
_NEG = float("-inf")
_I32MIN = -2147483648


def _bin_bounds(rois, H, W):
    rois_i = rois.astype(jnp.int32)
    batch_idx = rois_i[:, 0]
    roi_start_w = rois_i[:, 1].astype(jnp.float32)
    roi_start_h = rois_i[:, 2].astype(jnp.float32)
    roi_end_w = rois_i[:, 3].astype(jnp.float32)
    roi_end_h = rois_i[:, 4].astype(jnp.float32)
    roi_height = jnp.maximum(roi_end_h - roi_start_h, 1.0)
    roi_width = jnp.maximum(roi_end_w - roi_start_w, 1.0)
    bin_h = roi_height / float(_PH)
    bin_w = roi_width / float(_PW)
    hs = jnp.arange(_PH, dtype=jnp.float32)
    ws = jnp.arange(_PW, dtype=jnp.float32)
    h_start = jnp.floor(hs[None, :] * bin_h[:, None] + roi_start_h[:, None]).astype(jnp.int32)
    h_end = jnp.ceil((hs[None, :] + 1.0) * bin_h[:, None] + roi_start_h[:, None]).astype(jnp.int32)
    w_start = jnp.floor(ws[None, :] * bin_w[:, None] + roi_start_w[:, None]).astype(jnp.int32)
    w_end = jnp.ceil((ws[None, :] + 1.0) * bin_w[:, None] + roi_start_w[:, None]).astype(jnp.int32)
    h_start = jnp.clip(h_start, 0, H)
    h_end = jnp.clip(h_end, 0, H)
    w_start = jnp.clip(w_start, 0, W)
    w_end = jnp.clip(w_end, 0, W)
    return batch_idx, h_start, h_end, w_start, w_end


def _lane_i32(v, j):
    return jnp.max(jnp.where(lax.iota(jnp.int32, _LANES) == j, v, _I32MIN))


def _make_sc_kernel(B, H, W, C, N, NP):
    HW_C = H * W * C
    n_chunks = C // _LANES
    row = C * _NBINS
    mesh = plsc.VectorSubcoreMesh(
        core_axis_name="c", subcore_axis_name="s", num_cores=2, num_subcores=16
    )
    cp = pltpu.CompilerParams()
    if "needs_layout_passes" in pltpu.CompilerParams.__dataclass_fields__:
        cp = dataclasses.replace(cp, needs_layout_passes=False)

    @functools.partial(
        pl.kernel,
        out_type=jax.ShapeDtypeStruct(((N + 2) * row,), jnp.float32),
        mesh=mesh,
        compiler_params=cp,
        scratch_types=[
            pltpu.VMEM((HW_C,), jnp.float32),
            pltpu.VMEM((NP,), jnp.int32),
            pltpu.VMEM((32,), jnp.int32),
            pltpu.VMEM((2 * row,), jnp.float32),
            pltpu.VMEM((NP,), jnp.int32),
            pltpu.SemaphoreType.DMA,
        ],
    )
    def sc_kernel(
        feats_hbm, batch_hbm, packed_hbm, out_hbm, feat_v, batch_v, pk_v, out_v, list_v, sem
    ):
        w = lax.axis_index("c") * 16 + lax.axis_index("s")
        pltpu.sync_copy(feats_hbm.at[w], feat_v)
        pltpu.sync_copy(batch_hbm, batch_v)
        lane_iota = lax.iota(jnp.int32, _LANES)
        neg = jnp.full((_LANES,), _NEG, jnp.float32)

        dummy = pl.multiple_of(N * row, 8)

        def process_roi(r, i):
            pltpu.sync_copy(packed_hbm.at[r], pk_v)
            v0 = pk_v[pl.ds(0, _LANES)]
            v1 = pk_v[pl.ds(16, _LANES)]
            hs = [_lane_i32(v0, i) for i in range(_PH)]
            he = [_lane_i32(v0, _PH + i) for i in range(_PH)]
            ws = [_lane_i32(v1, i) for i in range(_PW)]
            we = [_lane_i32(v1, _PW + i) for i in range(_PW)]
            pltpu.make_async_copy(
                out_hbm.at[pl.ds(dummy, row)], out_v.at[pl.ds(0, row)], sem
            ).wait()
            off = (i % 2) * row
            for hb in range(_PH):
                for wb in range(_PW):
                    def yloop(y, accs):
                        def xloop(x, accs):
                            base = (y * W + x) * C
                            return [
                                jnp.maximum(a, feat_v[pl.ds(base + k * _LANES, _LANES)])
                                for k, a in enumerate(accs)
                            ]
                        return lax.fori_loop(ws[wb], we[wb], xloop, accs)
                    accs = lax.fori_loop(hs[hb], he[hb], yloop, [neg] * n_chunks)
                    for k in range(n_chunks):
                        val = jnp.where(accs[k] == neg, 0.0, accs[k])
                        idx = (lane_iota + k * _LANES) * _NBINS + (hb * _PW + wb) + off
                        plsc.store_scatter(out_v, [idx], val)
            dst = pl.multiple_of(r * row, 8)
            pltpu.async_copy(
                out_v.at[pl.ds(off, row)], out_hbm.at[pl.ds(dst, row)], sem
            )

        def group(g, cnt):
            bm = batch_v[pl.ds(g * _LANES, _LANES)]
            mask = bm == w
            prefix = plsc.cumsum(jnp.where(mask, 1, 0))
            pos = cnt + prefix - 1
            plsc.store_scatter(list_v, [pos], g * _LANES + lane_iota, mask=mask)
            return cnt + plsc.all_reduce_population_count(mask)

        cnt = lax.fori_loop(
            0, NP // _LANES, group, jnp.zeros((_LANES,), jnp.int32)
        )
        total = jnp.max(cnt)

        pltpu.async_copy(out_v.at[pl.ds(0, row)], out_hbm.at[pl.ds(dummy, row)], sem)
        pltpu.async_copy(
            out_v.at[pl.ds(row, row)], out_hbm.at[pl.ds(dummy + row, row)], sem
        )

        def per_roi(i, _):
            grp = (i // _LANES) * _LANES
            lane = i - grp
            v = list_v[pl.ds(grp, _LANES)]
            r = jnp.max(jnp.where(lane_iota == lane, v, _I32MIN))
            process_roi(r, i)
            return 0

        lax.fori_loop(0, total, per_roi, 0)

        pltpu.make_async_copy(
            out_hbm.at[pl.ds(dummy, row)], out_v.at[pl.ds(0, row)], sem
        ).wait()
        pltpu.make_async_copy(
            out_hbm.at[pl.ds(dummy, row)], out_v.at[pl.ds(0, row)], sem
        ).wait()

    return sc_kernel


@jax.jit
def kernel(features, rois):
    B, C, H, W = features.shape
    N = rois.shape[0]
    batch_idx, h_start, h_end, w_start, w_end = _bin_bounds(rois, H, W)
    featsT = jnp.transpose(features, (0, 2, 3, 1)).reshape(B, H * W * C)
    NP = ((N + _LANES - 1) // _LANES) * _LANES
    batch_p = jnp.full((NP,), 127, jnp.int32).at[:N].set(batch_idx)
    zeros2 = jnp.zeros((N, 2), jnp.int32)
    packed = jnp.concatenate(
        [h_start, h_end, zeros2, w_start, w_end, zeros2], axis=1
    )
    out = _make_sc_kernel(B, H, W, C, N, NP)(featsT, batch_p, packed)
    return out[: N * C * _PH * _PW].reshape(N, C, _PH, _PW)


# speedup vs baseline: 3.3210x; 3.3210x over previous
import dataclasses
import functools

import jax
import jax.numpy as jnp
from jax import lax
from jax.experimental import pallas as pl
from jax.experimental.pallas import tpu as pltpu
from jax.experimental.pallas import tpu_sc as plsc

_PH, _PW = 7, 7
_NBINS = _PH * _PW
_LANES = 16
_NWORKERS = 32
_NEG = float("-inf")
_I32MIN = -2147483648


def _bin_bounds(rois, H, W):
    rois_i = rois.astype(jnp.int32)
    batch_idx = rois_i[:, 0]
    roi_start_w = rois_i[:, 1].astype(jnp.float32)
    roi_start_h = rois_i[:, 2].astype(jnp.float32)
    roi_end_w = rois_i[:, 3].astype(jnp.float32)
    roi_end_h = rois_i[:, 4].astype(jnp.float32)
    roi_height = jnp.maximum(roi_end_h - roi_start_h, 1.0)
    roi_width = jnp.maximum(roi_end_w - roi_start_w, 1.0)
    bin_h = roi_height / float(_PH)
    bin_w = roi_width / float(_PW)
    hs = jnp.arange(_PH, dtype=jnp.float32)
    ws = jnp.arange(_PW, dtype=jnp.float32)
    h_start = jnp.floor(hs[None, :] * bin_h[:, None] + roi_start_h[:, None]).astype(jnp.int32)
    h_end = jnp.ceil((hs[None, :] + 1.0) * bin_h[:, None] + roi_start_h[:, None]).astype(jnp.int32)
    w_start = jnp.floor(ws[None, :] * bin_w[:, None] + roi_start_w[:, None]).astype(jnp.int32)
    w_end = jnp.ceil((ws[None, :] + 1.0) * bin_w[:, None] + roi_start_w[:, None]).astype(jnp.int32)
    h_start = jnp.clip(h_start, 0, H)
    h_end = jnp.clip(h_end, 0, H)
    w_start = jnp.clip(w_start, 0, W)
    w_end = jnp.clip(w_end, 0, W)
    return batch_idx, h_start, h_end, w_start, w_end


def _lane_i32(v, j):
    return jnp.max(jnp.where(lax.iota(jnp.int32, _LANES) == j, v, _I32MIN))


def _make_sc_kernel(B, H, W, C, N, NP):
    HW_C = H * W * C
    n_chunks = C // _LANES
    row = C * _NBINS
    mesh = plsc.VectorSubcoreMesh(
        core_axis_name="c", subcore_axis_name="s", num_cores=2, num_subcores=16
    )
    cp = pltpu.CompilerParams()
    if "needs_layout_passes" in pltpu.CompilerParams.__dataclass_fields__:
        cp = dataclasses.replace(cp, needs_layout_passes=False)

    @functools.partial(
        pl.kernel,
        out_type=jax.ShapeDtypeStruct((N, row), jnp.float32),
        mesh=mesh,
        compiler_params=cp,
        scratch_types=[
            pltpu.VMEM((HW_C,), jnp.float32),
            pltpu.VMEM((NP,), jnp.int32),
            pltpu.VMEM((32,), jnp.int32),
            pltpu.VMEM((row,), jnp.float32),
            pltpu.VMEM((NP,), jnp.int32),
        ],
    )
    def sc_kernel(
        feats_hbm, batch_hbm, packed_hbm, out_hbm, feat_v, batch_v, pk_v, out_v, list_v
    ):
        w = lax.axis_index("c") * 16 + lax.axis_index("s")
        pltpu.sync_copy(feats_hbm.at[w], feat_v)
        pltpu.sync_copy(batch_hbm, batch_v)
        lane_iota = lax.iota(jnp.int32, _LANES)
        neg = jnp.full((_LANES,), _NEG, jnp.float32)

        def process_roi(r):
            pltpu.sync_copy(packed_hbm.at[r], pk_v)
            v0 = pk_v[pl.ds(0, _LANES)]
            v1 = pk_v[pl.ds(16, _LANES)]
            hs = [_lane_i32(v0, i) for i in range(_PH)]
            he = [_lane_i32(v0, _PH + i) for i in range(_PH)]
            ws = [_lane_i32(v1, i) for i in range(_PW)]
            we = [_lane_i32(v1, _PW + i) for i in range(_PW)]
            for hb in range(_PH):
                for wb in range(_PW):
                    def yloop(y, accs):
                        def xloop(x, accs):
                            base = (y * W + x) * C
                            return [
                                jnp.maximum(a, feat_v[pl.ds(base + k * _LANES, _LANES)])
                                for k, a in enumerate(accs)
                            ]
                        return lax.fori_loop(ws[wb], we[wb], xloop, accs)
                    accs = lax.fori_loop(hs[hb], he[hb], yloop, [neg] * n_chunks)
                    for k in range(n_chunks):
                        val = jnp.where(accs[k] == neg, 0.0, accs[k])
                        idx = (lane_iota + k * _LANES) * _NBINS + (hb * _PW + wb)
                        plsc.store_scatter(out_v, [idx], val)
            pltpu.sync_copy(out_v, out_hbm.at[r])

        def group(g, cnt):
            bm = batch_v[pl.ds(g * _LANES, _LANES)]
            mask = bm == w
            prefix = plsc.cumsum(jnp.where(mask, 1, 0))
            pos = cnt + prefix - 1
            plsc.store_scatter(list_v, [pos], g * _LANES + lane_iota, mask=mask)
            return cnt + plsc.all_reduce_population_count(mask)

        cnt = lax.fori_loop(
            0, NP // _LANES, group, jnp.zeros((_LANES,), jnp.int32)
        )
        total = jnp.max(cnt)

        def per_roi(i, _):
            grp = (i // _LANES) * _LANES
            lane = i - grp
            v = list_v[pl.ds(grp, _LANES)]
            r = jnp.max(jnp.where(lane_iota == lane, v, _I32MIN))
            process_roi(r)
            return 0

        lax.fori_loop(0, total, per_roi, 0)

    return sc_kernel


_TC_ROIS_PER_STEP = 8
_N_TC = 224


def _tc_body(b_ref, hs_ref, he_ref, ws_ref, we_ref, feats_ref, out_ref):
    step = pl.program_id(0)
    H, W = feats_ref.shape[1], feats_ref.shape[2]
    iw = lax.broadcasted_iota(jnp.int32, (1, W, 1), 1)
    ih = lax.broadcasted_iota(jnp.int32, (1, H, 1), 1)
    for i in range(_TC_ROIS_PER_STEP):
        r = step * _TC_ROIS_PER_STEP + i
        feat = feats_ref[b_ref[r]]
        cols = []
        for wb in range(_PW):
            m = (iw >= ws_ref[r * _PW + wb]) & (iw < we_ref[r * _PW + wb])
            cols.append(jnp.max(jnp.where(m, feat, _NEG), axis=1))
        tmp = jnp.stack(cols, axis=0)
        rows = []
        for hb in range(_PH):
            m = (ih >= hs_ref[r * _PH + hb]) & (ih < he_ref[r * _PH + hb])
            rows.append(jnp.max(jnp.where(m, tmp, _NEG), axis=1))
        pooled = jnp.stack(rows, axis=0)
        out_ref[i] = jnp.where(jnp.isfinite(pooled), pooled, 0.0)


def _tc_pool(featsT4, batch_idx, h_start, h_end, w_start, w_end):
    B, H, W, C = featsT4.shape
    n = batch_idx.shape[0]
    grid_spec = pltpu.PrefetchScalarGridSpec(
        num_scalar_prefetch=5,
        grid=(n // _TC_ROIS_PER_STEP,),
        in_specs=[pl.BlockSpec((B, H, W, C), lambda r, *_: (0, 0, 0, 0))],
        out_specs=pl.BlockSpec(
            (_TC_ROIS_PER_STEP, _PH, _PW, C), lambda r, *_: (r, 0, 0, 0)
        ),
    )
    out = pl.pallas_call(
        _tc_body,
        grid_spec=grid_spec,
        out_shape=jax.ShapeDtypeStruct((n, _PH, _PW, C), jnp.float32),
    )(
        batch_idx,
        h_start.reshape(-1),
        h_end.reshape(-1),
        w_start.reshape(-1),
        w_end.reshape(-1),
        featsT4,
    )
    return jnp.transpose(out, (0, 3, 1, 2))


@jax.jit
def kernel(features, rois):
    B, C, H, W = features.shape
    N = rois.shape[0]
    batch_idx, h_start, h_end, w_start, w_end = _bin_bounds(rois, H, W)
    featsT4 = jnp.transpose(features, (0, 2, 3, 1))
    featsT = featsT4.reshape(B, H * W * C)
    n_sc = N - _N_TC
    NP = ((n_sc + _LANES - 1) // _LANES) * _LANES
    batch_p = jnp.full((NP,), 127, jnp.int32).at[:n_sc].set(batch_idx[:n_sc])
    zeros2 = jnp.zeros((n_sc, 2), jnp.int32)
    packed = jnp.concatenate(
        [h_start[:n_sc], h_end[:n_sc], zeros2, w_start[:n_sc], w_end[:n_sc], zeros2],
        axis=1,
    )
    out_sc = _make_sc_kernel(B, H, W, C, n_sc, NP)(featsT, batch_p, packed)
    out_tc = _tc_pool(
        featsT4,
        batch_idx[n_sc:],
        h_start[n_sc:],
        h_end[n_sc:],
        w_start[n_sc:],
        w_end[n_sc:],
    )
    return jnp.concatenate(
        [out_sc.reshape(n_sc, C, _PH, _PW), out_tc], axis=0
    )


# speedup vs baseline: 4.0601x; 1.2226x over previous
import dataclasses
import functools

import jax
import jax.numpy as jnp
from jax import lax
from jax.experimental import pallas as pl
from jax.experimental.pallas import tpu as pltpu
from jax.experimental.pallas import tpu_sc as plsc

_PH, _PW = 7, 7
_NBINS = _PH * _PW
_LANES = 16
_NWORKERS = 32
_NEG = float("-inf")
_I32MIN = -2147483648


def _bin_bounds(rois, H, W):
    rois_i = rois.astype(jnp.int32)
    batch_idx = rois_i[:, 0]
    roi_start_w = rois_i[:, 1].astype(jnp.float32)
    roi_start_h = rois_i[:, 2].astype(jnp.float32)
    roi_end_w = rois_i[:, 3].astype(jnp.float32)
    roi_end_h = rois_i[:, 4].astype(jnp.float32)
    roi_height = jnp.maximum(roi_end_h - roi_start_h, 1.0)
    roi_width = jnp.maximum(roi_end_w - roi_start_w, 1.0)
    bin_h = roi_height / float(_PH)
    bin_w = roi_width / float(_PW)
    hs = jnp.arange(_PH, dtype=jnp.float32)
    ws = jnp.arange(_PW, dtype=jnp.float32)
    h_start = jnp.floor(hs[None, :] * bin_h[:, None] + roi_start_h[:, None]).astype(jnp.int32)
    h_end = jnp.ceil((hs[None, :] + 1.0) * bin_h[:, None] + roi_start_h[:, None]).astype(jnp.int32)
    w_start = jnp.floor(ws[None, :] * bin_w[:, None] + roi_start_w[:, None]).astype(jnp.int32)
    w_end = jnp.ceil((ws[None, :] + 1.0) * bin_w[:, None] + roi_start_w[:, None]).astype(jnp.int32)
    h_start = jnp.clip(h_start, 0, H)
    h_end = jnp.clip(h_end, 0, H)
    w_start = jnp.clip(w_start, 0, W)
    w_end = jnp.clip(w_end, 0, W)
    return batch_idx, h_start, h_end, w_start, w_end


def _lane_i32(v, j):
    return jnp.max(jnp.where(lax.iota(jnp.int32, _LANES) == j, v, _I32MIN))


def _make_sc_kernel(B, H, W, C, N, NP):
    HW_C = H * W * C
    n_chunks = C // _LANES
    row = C * _NBINS
    mesh = plsc.VectorSubcoreMesh(
        core_axis_name="c", subcore_axis_name="s", num_cores=2, num_subcores=16
    )
    cp = pltpu.CompilerParams()
    if "needs_layout_passes" in pltpu.CompilerParams.__dataclass_fields__:
        cp = dataclasses.replace(cp, needs_layout_passes=False)

    @functools.partial(
        pl.kernel,
        out_type=jax.ShapeDtypeStruct((N, row), jnp.float32),
        mesh=mesh,
        compiler_params=cp,
        scratch_types=[
            pltpu.VMEM((HW_C,), jnp.float32),
            pltpu.VMEM((NP,), jnp.int32),
            pltpu.VMEM((32,), jnp.int32),
            pltpu.VMEM((row,), jnp.float32),
            pltpu.VMEM((NP,), jnp.int32),
        ],
    )
    def sc_kernel(
        feats_hbm, batch_hbm, packed_hbm, out_hbm, feat_v, batch_v, pk_v, out_v, list_v
    ):
        w = lax.axis_index("c") * 16 + lax.axis_index("s")
        pltpu.sync_copy(feats_hbm.at[w], feat_v)
        pltpu.sync_copy(batch_hbm, batch_v)
        lane_iota = lax.iota(jnp.int32, _LANES)
        neg = jnp.full((_LANES,), _NEG, jnp.float32)

        def process_roi(r):
            pltpu.sync_copy(packed_hbm.at[r], pk_v)
            v0 = pk_v[pl.ds(0, _LANES)]
            v1 = pk_v[pl.ds(16, _LANES)]
            hs = [_lane_i32(v0, i) for i in range(_PH)]
            he = [_lane_i32(v0, _PH + i) for i in range(_PH)]
            ws = [_lane_i32(v1, i) for i in range(_PW)]
            we = [_lane_i32(v1, _PW + i) for i in range(_PW)]
            for hb in range(_PH):
                for wb in range(_PW):
                    def yloop(y, accs):
                        def xloop(x, accs):
                            base = (y * W + x) * C
                            return [
                                jnp.maximum(a, feat_v[pl.ds(base + k * _LANES, _LANES)])
                                for k, a in enumerate(accs)
                            ]
                        return lax.fori_loop(ws[wb], we[wb], xloop, accs)
                    accs = lax.fori_loop(hs[hb], he[hb], yloop, [neg] * n_chunks)
                    for k in range(n_chunks):
                        val = jnp.where(accs[k] == neg, 0.0, accs[k])
                        idx = (lane_iota + k * _LANES) * _NBINS + (hb * _PW + wb)
                        plsc.store_scatter(out_v, [idx], val)
            pltpu.sync_copy(out_v, out_hbm.at[r])

        def group(g, cnt):
            bm = batch_v[pl.ds(g * _LANES, _LANES)]
            mask = bm == w
            prefix = plsc.cumsum(jnp.where(mask, 1, 0))
            pos = cnt + prefix - 1
            plsc.store_scatter(list_v, [pos], g * _LANES + lane_iota, mask=mask)
            return cnt + plsc.all_reduce_population_count(mask)

        cnt = lax.fori_loop(
            0, NP // _LANES, group, jnp.zeros((_LANES,), jnp.int32)
        )
        total = jnp.max(cnt)

        def per_roi(i, _):
            grp = (i // _LANES) * _LANES
            lane = i - grp
            v = list_v[pl.ds(grp, _LANES)]
            r = jnp.max(jnp.where(lane_iota == lane, v, _I32MIN))
            process_roi(r)
            return 0

        lax.fori_loop(0, total, per_roi, 0)

    return sc_kernel


@jax.jit
def kernel(features, rois):
    B, C, H, W = features.shape
    N = rois.shape[0]
    batch_idx, h_start, h_end, w_start, w_end = _bin_bounds(rois, H, W)
    featsT = jnp.transpose(features, (0, 2, 3, 1)).reshape(B, H * W * C)
    NP = ((N + _LANES - 1) // _LANES) * _LANES
    batch_p = jnp.full((NP,), 127, jnp.int32).at[:N].set(batch_idx)
    zeros2 = jnp.zeros((N, 2), jnp.int32)
    packed = jnp.concatenate(
        [h_start, h_end, zeros2, w_start, w_end, zeros2], axis=1
    )
    out = _make_sc_kernel(B, H, W, C, N, NP)(featsT, batch_p, packed)
    return out.reshape(N, C, _PH, _PW)
